# baseline (speedup 1.0000x reference)
import dataclasses
import functools

import jax
import jax.numpy as jnp
from jax import lax
from jax.experimental import pallas as pl
from jax.experimental.pallas import tpu as pltpu
from jax.experimental.pallas import tpu_sc as plsc

N = 10000
D = 128
E = 320000
LANES = 16

NC = 2
NS = 16
NW = NC * NS

CHUNK = 32
CHUNKS_PER_SUP = 16
NBUF = 8
GA = 6
SUP = CHUNK * CHUNKS_PER_SUP
NSUP = 20
NSUP_PAD = 21
EPT = SUP * NSUP
E_PAD = EPT * NW

ROWS_PER_TILE = 624


def _spmm_sc(x, src3, dst3, w3):
    mesh = plsc.VectorSubcoreMesh(core_axis_name="c", subcore_axis_name="s")
    cp = pltpu.CompilerParams()
    if "needs_layout_passes" in pltpu.CompilerParams.__dataclass_fields__:
        cp = dataclasses.replace(cp, needs_layout_passes=False)

    @functools.partial(
        pl.kernel,
        compiler_params=cp,
        out_type=jax.ShapeDtypeStruct((NC, N, D), jnp.float32),
        mesh=mesh,
        scratch_types=[
            pltpu.VMEM((CHUNKS_PER_SUP, CHUNK), jnp.int32),
            pltpu.VMEM((CHUNKS_PER_SUP, CHUNK), jnp.int32),
            pltpu.VMEM((CHUNKS_PER_SUP, CHUNK), jnp.int32),
            pltpu.VMEM((CHUNKS_PER_SUP, CHUNK), jnp.int32),
            pltpu.VMEM((SUP,), jnp.float32),
            pltpu.VMEM((SUP,), jnp.float32),
        ] + [pltpu.VMEM((CHUNK, D), jnp.float32)] * NBUF
          + [pltpu.VMEM_SHARED((N, D), jnp.float32)]
          + [pltpu.SemaphoreType.DMA] * (2 * NBUF + 2),
    )
    def spmm_kernel(x_hbm, src_hbm, dst_hbm, w_hbm, out_hbm, *refs):
        sidx = refs[0:2]
        didx = refs[2:4]
        wbuf = refs[4:6]
        rows = refs[6:6 + NBUF]
        acc = refs[6 + NBUF]
        gsems = refs[7 + NBUF:7 + 2 * NBUF]
        ssems = refs[7 + 2 * NBUF:7 + 3 * NBUF]
        tsems = refs[7 + 3 * NBUF:7 + 3 * NBUF + 2]
        cid = lax.axis_index("c")
        sid = lax.axis_index("s")
        wid = sid * NC + cid
        CPS = CHUNKS_PER_SUP

        zero16 = jnp.zeros((LANES,), jnp.float32)

        @pl.loop(0, CHUNK)
        def _(r):
            for k in range(D // LANES):
                sl = pl.ds(k * LANES, LANES)
                rows[0][r, sl] = zero16
                rows[NBUF - 2][r, sl] = zero16
                rows[NBUF - 1][r, sl] = zero16

        base = pl.multiple_of(sid * ROWS_PER_TILE, 8)
        nfull = ROWS_PER_TILE // CHUNK
        tail = ROWS_PER_TILE - nfull * CHUNK
        zdescs = [pltpu.make_async_copy(
            rows[0].at[pl.ds(0, CHUNK)],
            acc.at[pl.ds(base + b * CHUNK, CHUNK)], gsems[0])
            for b in range(nfull)]
        if tail:
            zdescs.append(pltpu.make_async_copy(
                rows[0].at[pl.ds(0, tail)],
                acc.at[pl.ds(base + nfull * CHUNK, tail)], gsems[0]))
        for d in zdescs:
            d.start()
        for d in zdescs:
            d.wait()

        @pl.when(sid == NS - 1)
        def _():
            pltpu.sync_copy(rows[0].at[pl.ds(0, N - NS * ROWS_PER_TILE)],
                            acc.at[pl.ds(NS * ROWS_PER_TILE,
                                         N - NS * ROWS_PER_TILE)])

        plsc.subcore_barrier()

        def stage_descs(s, p):
            return (pltpu.make_async_copy(src_hbm.at[wid, s], sidx[p],
                                          tsems[p]),
                    pltpu.make_async_copy(dst_hbm.at[wid, s], didx[p],
                                          tsems[p]),
                    pltpu.make_async_copy(w_hbm.at[wid, s], wbuf[p],
                                          tsems[p]))

        def gather_desc(p, j, b):
            return pltpu.make_async_copy(x_hbm.at[sidx[p].at[j]],
                                         rows[b], gsems[b])

        def scatter_desc(p, j, b):
            return pltpu.make_async_copy(rows[b],
                                         acc.at[didx[p].at[j]],
                                         ssems[b])

        for d in stage_descs(0, 0):
            d.start()
        for d in stage_descs(0, 0):
            d.wait()
        for c in range(GA):
            gather_desc(0, c, c).start()
        scatter_desc(0, 0, NBUF - 2).start(add=True)
        scatter_desc(0, 0, NBUF - 1).start(add=True)

        @pl.loop(0, NSUP // 2)
        def _(t):
            s0 = 2 * t
            for cc in range(2 * CPS):
                p, j = divmod(cc, CPS)
                b = cc % NBUF
                if cc == 2:
                    for d in stage_descs(s0 + 1, 1):
                        d.start()
                if cc == CPS + 2:
                    for d in stage_descs(s0 + 2, 0):
                        d.start()
                gather_desc(p, j, b).wait()

                @pl.loop(0, CHUNK, unroll=2)
                def _(e, j=j, p=p, b=b):
                    widx = jnp.full((LANES,), j * CHUNK, jnp.int32) + e
                    wv = plsc.load_gather(wbuf[p], [widx])
                    for k in range(D // LANES):
                        sl = pl.ds(k * LANES, LANES)
                        rows[b][e, sl] = rows[b][e, sl] * wv

                scatter_desc(p, j, b).start(add=True)
                pw, jw = divmod((cc - 2) % (2 * CPS), CPS)
                scatter_desc(pw, jw, (cc - 2) % NBUF).wait()
                if cc == CPS - GA:
                    for d in stage_descs(s0 + 1, 1):
                        d.wait()
                if cc == 2 * CPS - GA:
                    for d in stage_descs(s0 + 2, 0):
                        d.wait()
                p2, j2 = divmod((cc + GA) % (2 * CPS), CPS)
                gather_desc(p2, j2, (cc + GA) % NBUF).start()

        scatter_desc(1, CPS - 2, (2 * CPS - 2) % NBUF).wait()
        scatter_desc(1, CPS - 1, (2 * CPS - 1) % NBUF).wait()
        for c in range(GA):
            gather_desc(0, c, c).wait()

        plsc.subcore_barrier()

        odescs = [pltpu.make_async_copy(
            acc.at[pl.ds(base + b * CHUNK, CHUNK)],
            out_hbm.at[cid, pl.ds(base + b * CHUNK, CHUNK)], gsems[1])
            for b in range(nfull)]
        if tail:
            odescs.append(pltpu.make_async_copy(
                acc.at[pl.ds(base + nfull * CHUNK, tail)],
                out_hbm.at[cid, pl.ds(base + nfull * CHUNK, tail)],
                gsems[1]))
        for d in odescs:
            d.start()
        for d in odescs:
            d.wait()

        @pl.when(sid == NS - 1)
        def _():
            pltpu.sync_copy(acc.at[pl.ds(NS * ROWS_PER_TILE,
                                         N - NS * ROWS_PER_TILE)],
                            out_hbm.at[cid, pl.ds(NS * ROWS_PER_TILE,
                                                  N - NS * ROWS_PER_TILE)])

    return spmm_kernel(x, src3, dst3, w3)


BN = 1000


def _dense_body(ego_ref, p_ref, ws_ref, bs_ref, wb_ref, bb_ref,
                next_ref, norm_ref):
    ego = ego_ref[...]
    side = p_ref[0] + p_ref[1]
    h1 = jnp.dot(ego + side, ws_ref[...],
                 preferred_element_type=jnp.float32) + bs_ref[...]
    h2 = jnp.dot(ego * side, wb_ref[...],
                 preferred_element_type=jnp.float32) + bb_ref[...]
    h1 = jnp.where(h1 >= 0, h1, 0.01 * h1)
    h2 = jnp.where(h2 >= 0, h2, 0.01 * h2)
    nxt = h1 + h2
    next_ref[...] = nxt
    nrm = jnp.sqrt(jnp.sum(nxt * nxt, axis=1, keepdims=True))
    norm_ref[...] = nxt / jnp.maximum(nrm, 1e-12)


def _dense_tc(ego, parts, Ws, bs, Wb, bb):
    return pl.pallas_call(
        _dense_body,
        grid=(N // BN,),
        in_specs=[
            pl.BlockSpec((BN, D), lambda i: (i, 0)),
            pl.BlockSpec((NC, BN, D), lambda i: (0, i, 0)),
            pl.BlockSpec((D, D), lambda i: (0, 0)),
            pl.BlockSpec((1, D), lambda i: (0, 0)),
            pl.BlockSpec((D, D), lambda i: (0, 0)),
            pl.BlockSpec((1, D), lambda i: (0, 0)),
        ],
        out_specs=(pl.BlockSpec((BN, D), lambda i: (i, 0)),
                   pl.BlockSpec((BN, D), lambda i: (i, 0))),
        out_shape=(jax.ShapeDtypeStruct((N, D), jnp.float32),
                   jax.ShapeDtypeStruct((N, D), jnp.float32)),
    )(ego, parts, Ws, bs, Wb, bb)


def kernel(edge_index, edge_weight, embeddings,
           W_sum0, b_sum0, W_bi0, b_bi0,
           W_sum1, b_sum1, W_bi1, b_bi1,
           W_sum2, b_sum2, W_bi2, b_bi2):
    pad = E_PAD - E

    def _lay_out(v, dtype):
        v = jnp.concatenate([v, jnp.zeros((pad,), dtype)]).reshape(NW, NSUP, SUP)
        v = jnp.concatenate([v, jnp.zeros((NW, NSUP_PAD - NSUP, SUP), dtype)],
                            axis=1)
        return v

    src3 = _lay_out(edge_index[0], jnp.int32).reshape(
        NW, NSUP_PAD, CHUNKS_PER_SUP, CHUNK)
    dst3 = _lay_out(edge_index[1], jnp.int32).reshape(
        NW, NSUP_PAD, CHUNKS_PER_SUP, CHUNK)
    w3 = _lay_out(edge_weight, jnp.float32)

    params = [(W_sum0, b_sum0, W_bi0, b_bi0),
              (W_sum1, b_sum1, W_bi1, b_bi1),
              (W_sum2, b_sum2, W_bi2, b_bi2)]
    outs = [embeddings]
    ego = embeddings
    for (Ws, bs, Wb, bb) in params:
        parts = _spmm_sc(ego, src3, dst3, w3)
        ego, nrm = _dense_tc(ego, parts,
                             Ws, bs.reshape(1, D), Wb, bb.reshape(1, D))
        outs.append(nrm)
    return tuple(outs)


# speedup vs baseline: 1.1409x; 1.1409x over previous
import dataclasses
import functools

import jax
import jax.numpy as jnp
from jax import lax
from jax.experimental import pallas as pl
from jax.experimental.pallas import tpu as pltpu
from jax.experimental.pallas import tpu_sc as plsc

N = 10000
D = 128
E = 320000
LANES = 16

NC = 2
NS = 16
NW = NC * NS

CHUNK = 64
CHUNKS_PER_SUP = 16
SUP = CHUNK * CHUNKS_PER_SUP
NSUP = 10
NSUP_PAD = 11
EPT = SUP * NSUP
E_PAD = EPT * NW

ROWS_PER_TILE = 624


def _spmm_sc(x, src3, dst3, w3):
    mesh = plsc.VectorSubcoreMesh(core_axis_name="c", subcore_axis_name="s")
    cp = pltpu.CompilerParams()
    if "needs_layout_passes" in pltpu.CompilerParams.__dataclass_fields__:
        cp = dataclasses.replace(cp, needs_layout_passes=False)

    @functools.partial(
        pl.kernel,
        compiler_params=cp,
        out_type=jax.ShapeDtypeStruct((NC, N, D), jnp.float32),
        mesh=mesh,
        scratch_types=[
            pltpu.VMEM((CHUNKS_PER_SUP, CHUNK), jnp.int32),
            pltpu.VMEM((CHUNKS_PER_SUP, CHUNK), jnp.int32),
            pltpu.VMEM((CHUNKS_PER_SUP, CHUNK), jnp.int32),
            pltpu.VMEM((CHUNKS_PER_SUP, CHUNK), jnp.int32),
            pltpu.VMEM((SUP,), jnp.float32),
            pltpu.VMEM((SUP,), jnp.float32),
            pltpu.VMEM((CHUNK, D), jnp.float32),
            pltpu.VMEM((CHUNK, D), jnp.float32),
            pltpu.VMEM((CHUNK, D), jnp.float32),
            pltpu.VMEM((CHUNK, D), jnp.float32),
            pltpu.VMEM_SHARED((N, D), jnp.float32),
        ] + [pltpu.SemaphoreType.DMA] * 10,
    )
    def spmm_kernel(x_hbm, src_hbm, dst_hbm, w_hbm, out_hbm,
                    sidx0, sidx1, didx0, didx1, wbuf0, wbuf1,
                    rows0, rows1, rows2, rows3,
                    acc,
                    gsem0, gsem1, gsem2, gsem3,
                    ssem0, ssem1, ssem2, ssem3, tsem0, tsem1):
        cid = lax.axis_index("c")
        sid = lax.axis_index("s")
        wid = sid * NC + cid
        sidx = (sidx0, sidx1)
        didx = (didx0, didx1)
        wbuf = (wbuf0, wbuf1)
        rows = (rows0, rows1, rows2, rows3)
        gsems = (gsem0, gsem1, gsem2, gsem3)
        ssems = (ssem0, ssem1, ssem2, ssem3)
        tsems = (tsem0, tsem1)

        zero16 = jnp.zeros((LANES,), jnp.float32)

        @pl.loop(0, CHUNK)
        def _(r):
            for k in range(D // LANES):
                sl = pl.ds(k * LANES, LANES)
                rows0[r, sl] = zero16
                rows2[r, sl] = zero16
                rows3[r, sl] = zero16

        base = pl.multiple_of(sid * ROWS_PER_TILE, 8)
        nfull = ROWS_PER_TILE // CHUNK
        tail = ROWS_PER_TILE - nfull * CHUNK
        for b in range(nfull):
            pltpu.sync_copy(rows0.at[pl.ds(0, CHUNK)],
                            acc.at[pl.ds(base + b * CHUNK, CHUNK)])
        pltpu.sync_copy(rows0.at[pl.ds(0, tail)],
                        acc.at[pl.ds(base + nfull * CHUNK, tail)])

        @pl.when(sid == NS - 1)
        def _():
            pltpu.sync_copy(rows0.at[pl.ds(0, N - NS * ROWS_PER_TILE)],
                            acc.at[pl.ds(NS * ROWS_PER_TILE,
                                         N - NS * ROWS_PER_TILE)])

        plsc.subcore_barrier()

        def stage_descs(s, p):
            return (pltpu.make_async_copy(src_hbm.at[wid, s], sidx[p],
                                          tsems[p]),
                    pltpu.make_async_copy(dst_hbm.at[wid, s], didx[p],
                                          tsems[p]),
                    pltpu.make_async_copy(w_hbm.at[wid, s], wbuf[p],
                                          tsems[p]))

        def gather_desc(p, j, b):
            return pltpu.make_async_copy(x_hbm.at[sidx[p].at[j]],
                                         rows[b], gsems[b])

        def scatter_desc(p, j, b):
            return pltpu.make_async_copy(rows[b],
                                         acc.at[didx[p].at[j]],
                                         ssems[b])

        for d in stage_descs(0, 0):
            d.start()
        for d in stage_descs(0, 0):
            d.wait()
        gather_desc(0, 0, 0).start()
        gather_desc(0, 1, 1).start()
        scatter_desc(0, 0, 2).start(add=True)
        scatter_desc(0, 0, 3).start(add=True)

        @pl.loop(0, NSUP // 2)
        def _(t):
            s0 = 2 * t
            for cc in range(2 * CHUNKS_PER_SUP):
                p, j = divmod(cc, CHUNKS_PER_SUP)
                b = cc % 4
                if cc == 2:
                    for d in stage_descs(s0 + 1, 1):
                        d.start()
                if cc == CHUNKS_PER_SUP + 2:
                    for d in stage_descs(s0 + 2, 0):
                        d.start()
                gather_desc(p, j, b).wait()

                @pl.loop(0, CHUNK, unroll=2)
                def _(e, j=j, p=p, b=b):
                    widx = jnp.full((LANES,), j * CHUNK, jnp.int32) + e
                    wv = plsc.load_gather(wbuf[p], [widx])
                    for k in range(D // LANES):
                        sl = pl.ds(k * LANES, LANES)
                        rows[b][e, sl] = rows[b][e, sl] * wv

                scatter_desc(p, j, b).start(add=True)
                cc2 = cc + 2
                p2, j2 = divmod(cc2 % (2 * CHUNKS_PER_SUP), CHUNKS_PER_SUP)
                scatter_desc(p2, j2, cc2 % 4).wait()
                if cc == CHUNKS_PER_SUP - 2:
                    for d in stage_descs(s0 + 1, 1):
                        d.wait()
                if cc == 2 * CHUNKS_PER_SUP - 2:
                    for d in stage_descs(s0 + 2, 0):
                        d.wait()
                gather_desc(p2, j2, cc2 % 4).start()

        scatter_desc(1, CHUNKS_PER_SUP - 2, 2).wait()
        scatter_desc(1, CHUNKS_PER_SUP - 1, 3).wait()
        gather_desc(0, 0, 0).wait()
        gather_desc(0, 1, 1).wait()

        plsc.subcore_barrier()

        for b in range(nfull):
            pltpu.sync_copy(acc.at[pl.ds(base + b * CHUNK, CHUNK)],
                            out_hbm.at[cid, pl.ds(base + b * CHUNK, CHUNK)])
        pltpu.sync_copy(acc.at[pl.ds(base + nfull * CHUNK, tail)],
                        out_hbm.at[cid, pl.ds(base + nfull * CHUNK, tail)])

        @pl.when(sid == NS - 1)
        def _():
            pltpu.sync_copy(acc.at[pl.ds(NS * ROWS_PER_TILE,
                                         N - NS * ROWS_PER_TILE)],
                            out_hbm.at[cid, pl.ds(NS * ROWS_PER_TILE,
                                                  N - NS * ROWS_PER_TILE)])

    return spmm_kernel(x, src3, dst3, w3)


BN = 1000


def _dense_body(ego_ref, p_ref, ws_ref, bs_ref, wb_ref, bb_ref,
                next_ref, norm_ref):
    ego = ego_ref[...]
    side = p_ref[0] + p_ref[1]
    h1 = jnp.dot(ego + side, ws_ref[...],
                 preferred_element_type=jnp.float32) + bs_ref[...]
    h2 = jnp.dot(ego * side, wb_ref[...],
                 preferred_element_type=jnp.float32) + bb_ref[...]
    h1 = jnp.where(h1 >= 0, h1, 0.01 * h1)
    h2 = jnp.where(h2 >= 0, h2, 0.01 * h2)
    nxt = h1 + h2
    next_ref[...] = nxt
    nrm = jnp.sqrt(jnp.sum(nxt * nxt, axis=1, keepdims=True))
    norm_ref[...] = nxt / jnp.maximum(nrm, 1e-12)


def _dense_tc(ego, parts, Ws, bs, Wb, bb):
    return pl.pallas_call(
        _dense_body,
        grid=(N // BN,),
        in_specs=[
            pl.BlockSpec((BN, D), lambda i: (i, 0)),
            pl.BlockSpec((NC, BN, D), lambda i: (0, i, 0)),
            pl.BlockSpec((D, D), lambda i: (0, 0)),
            pl.BlockSpec((1, D), lambda i: (0, 0)),
            pl.BlockSpec((D, D), lambda i: (0, 0)),
            pl.BlockSpec((1, D), lambda i: (0, 0)),
        ],
        out_specs=(pl.BlockSpec((BN, D), lambda i: (i, 0)),
                   pl.BlockSpec((BN, D), lambda i: (i, 0))),
        out_shape=(jax.ShapeDtypeStruct((N, D), jnp.float32),
                   jax.ShapeDtypeStruct((N, D), jnp.float32)),
    )(ego, parts, Ws, bs, Wb, bb)


def kernel(edge_index, edge_weight, embeddings,
           W_sum0, b_sum0, W_bi0, b_bi0,
           W_sum1, b_sum1, W_bi1, b_bi1,
           W_sum2, b_sum2, W_bi2, b_bi2):
    pad = E_PAD - E

    def _lay_out(v, dtype):
        v = jnp.concatenate([v, jnp.zeros((pad,), dtype)]).reshape(NW, NSUP, SUP)
        v = jnp.concatenate([v, jnp.zeros((NW, NSUP_PAD - NSUP, SUP), dtype)],
                            axis=1)
        return v

    src3 = _lay_out(edge_index[0], jnp.int32).reshape(
        NW, NSUP_PAD, CHUNKS_PER_SUP, CHUNK)
    dst3 = _lay_out(edge_index[1], jnp.int32).reshape(
        NW, NSUP_PAD, CHUNKS_PER_SUP, CHUNK)
    w3 = _lay_out(edge_weight, jnp.float32)

    params = [(W_sum0, b_sum0, W_bi0, b_bi0),
              (W_sum1, b_sum1, W_bi1, b_bi1),
              (W_sum2, b_sum2, W_bi2, b_bi2)]
    outs = [embeddings]
    ego = embeddings
    for (Ws, bs, Wb, bb) in params:
        parts = _spmm_sc(ego, src3, dst3, w3)
        ego, nrm = _dense_tc(ego, parts,
                             Ws, bs.reshape(1, D), Wb, bb.reshape(1, D))
        outs.append(nrm)
    return tuple(outs)
